# baseline (speedup 1.0000x reference)
import functools

import jax
import jax.numpy as jnp
from jax import lax
from jax.experimental import pallas as pl
from jax.experimental.pallas import tpu as pltpu
from jax.experimental.pallas import tpu_sc as plsc

NUM_EXPERTS = 64
K = 8
NUM_SHARED = 2
K_SELECT = K - NUM_SHARED
END_IDX = NUM_EXPERTS - NUM_SHARED

BATCH = 32768
NC = 2
NS = 16
L = 16
NW = NC * NS
ROWS_PER_W = BATCH // NW
GROUPS = ROWS_PER_W // L
NSTREAM = 2
CHUNK_ROWS = 256
CHUNKS = ROWS_PER_W // CHUNK_ROWS
GROUPS_PER_CHUNK = CHUNK_ROWS // L

_mesh = plsc.VectorSubcoreMesh(
    core_axis_name="c", subcore_axis_name="s", num_cores=NC, num_subcores=NS)


@functools.partial(
    pl.kernel,
    out_type=[
        jax.ShapeDtypeStruct((BATCH * K,), jnp.int32),
        jax.ShapeDtypeStruct((BATCH * K,), jnp.float32),
    ],
    mesh=_mesh,
    scratch_types=[
        pltpu.VMEM((CHUNK_ROWS, NUM_EXPERTS), jnp.float32),
        pltpu.VMEM((ROWS_PER_W * K,), jnp.int32),
        pltpu.VMEM((ROWS_PER_W * K,), jnp.float32),
    ],
    compiler_params=pltpu.CompilerParams(needs_layout_passes=False),
)
def _sc_topk(gates_hbm, idx_hbm, gate_hbm, buf, oidx, ogate):
    wid = lax.axis_index("s") * NC + lax.axis_index("c")
    row_base = wid * ROWS_PER_W
    out_base = wid * (ROWS_PER_W * K)

    lane = lax.iota(jnp.int32, L)
    lane_out = lane * K
    neg_inf = jnp.full((L,), -jnp.inf, dtype=jnp.float32)
    i62 = jnp.full((L,), END_IDX, dtype=jnp.int32)
    i63 = jnp.full((L,), END_IDX + 1, dtype=jnp.int32)

    def chunk_body(ch, carry):
        pltpu.sync_copy(
            gates_hbm.at[pl.ds(row_base + ch * CHUNK_ROWS, CHUNK_ROWS), :], buf)
        lax.fori_loop(0, GROUPS_PER_CHUNK // NSTREAM,
                      functools.partial(group_body, ch), 0)
        return carry

    def group_body(ch, g, carry):
        rows = [g * (L * NSTREAM) + s * L + lane for s in range(NSTREAM)]
        obases = [(ch * GROUPS_PER_CHUNK + g * NSTREAM + s) * (L * K)
                  + lane_out for s in range(NSTREAM)]

        t = [[neg_inf] * K_SELECT for _ in range(NSTREAM)]
        ti = [[jnp.zeros((L,), jnp.int32)] * K_SELECT for _ in range(NSTREAM)]
        for j in range(END_IDX):
            cj = jnp.full((L,), j, dtype=jnp.int32)
            v = [plsc.load_gather(buf, [rows[s], cj]) for s in range(NSTREAM)]
            ci = [cj] * NSTREAM
            for lvl in range(K_SELECT):
                for s in range(NSTREAM):
                    hi = jnp.maximum(v[s], t[s][lvl])
                    lo = jnp.minimum(v[s], t[s][lvl])
                    c = v[s] > t[s][lvl]
                    ni = jnp.where(c, ci[s], ti[s][lvl])
                    ci[s] = jnp.where(c, ti[s][lvl], ci[s])
                    t[s][lvl], ti[s][lvl] = hi, ni
                    v[s] = lo

        for s in range(NSTREAM):
            s62 = plsc.load_gather(buf, [rows[s], i62])
            s63 = plsc.load_gather(buf, [rows[s], i63])

            vals = t[s] + [s62, s63]
            idxs = ti[s] + [i62, i63]

            m = jnp.maximum(jnp.maximum(t[s][0], s62), s63)
            es = [jnp.exp(x - m) for x in vals]
            total = ((es[0] + es[1]) + (es[2] + es[3])) + (
                (es[4] + es[5]) + (es[6] + es[7]))
            r = 1.0 / total

            for k in range(K):
                pos = obases[s] + k
                plsc.store_scatter(oidx, [pos], idxs[k])
                plsc.store_scatter(ogate, [pos], es[k] * r)
        return carry

    lax.fori_loop(0, CHUNKS, chunk_body, 0)

    pltpu.sync_copy(oidx, idx_hbm.at[pl.ds(out_base, ROWS_PER_W * K)])
    pltpu.sync_copy(ogate, gate_hbm.at[pl.ds(out_base, ROWS_PER_W * K)])


@jax.jit
def kernel(gates):
    batch = gates.shape[0]
    idx_flat, gate_flat = _sc_topk(gates)
    return idx_flat.reshape(batch, K), gate_flat.reshape(batch, K)


# speedup vs baseline: 1.0682x; 1.0682x over previous
import functools

import jax
import jax.numpy as jnp
from jax import lax
from jax.experimental import pallas as pl
from jax.experimental.pallas import tpu as pltpu
from jax.experimental.pallas import tpu_sc as plsc

NUM_EXPERTS = 64
K = 8
NUM_SHARED = 2
K_SELECT = K - NUM_SHARED
END_IDX = NUM_EXPERTS - NUM_SHARED

BATCH = 32768
NC = 2
NS = 16
L = 16
NW = NC * NS
ROWS_PER_W = BATCH // NW
CHUNK_ROWS = 256
CHUNKS = ROWS_PER_W // CHUNK_ROWS
GROUPS_PER_CHUNK = CHUNK_ROWS // L

_mesh = plsc.VectorSubcoreMesh(
    core_axis_name="c", subcore_axis_name="s", num_cores=NC, num_subcores=NS)


@functools.partial(
    pl.kernel,
    out_type=[
        jax.ShapeDtypeStruct((BATCH, K), jnp.int32),
        jax.ShapeDtypeStruct((BATCH, K), jnp.float32),
    ],
    mesh=_mesh,
    scratch_types=[
        pltpu.VMEM((CHUNK_ROWS, NUM_EXPERTS), jnp.float32),
        pltpu.VMEM((CHUNK_ROWS, K), jnp.int32),
        pltpu.VMEM((CHUNK_ROWS, K), jnp.float32),
    ],
    compiler_params=pltpu.CompilerParams(needs_layout_passes=False),
)
def _sc_topk(gates_hbm, idx_hbm, gate_hbm, buf, oidx, ogate):
    wid = lax.axis_index("s") * NC + lax.axis_index("c")
    row_base = wid * ROWS_PER_W

    lane = lax.iota(jnp.int32, L)
    neg_inf = jnp.full((L,), -jnp.inf, dtype=jnp.float32)
    i62 = jnp.full((L,), END_IDX, dtype=jnp.int32)
    i63 = jnp.full((L,), END_IDX + 1, dtype=jnp.int32)

    def group_body(g, carry):
        row = g * L + lane

        t = [neg_inf] * K_SELECT
        ti = [jnp.zeros((L,), jnp.int32)] * K_SELECT
        for j in range(END_IDX):
            cj = jnp.full((L,), j, dtype=jnp.int32)
            v = plsc.load_gather(buf, [row, cj])
            ci = cj
            for lvl in range(K_SELECT):
                hi = jnp.maximum(v, t[lvl])
                lo = jnp.minimum(v, t[lvl])
                c = v > t[lvl]
                ni = jnp.where(c, ci, ti[lvl])
                ci = jnp.where(c, ti[lvl], ci)
                t[lvl], ti[lvl] = hi, ni
                v = lo

        s62 = plsc.load_gather(buf, [row, i62])
        s63 = plsc.load_gather(buf, [row, i63])

        vals = t + [s62, s63]
        idxs = ti + [i62, i63]

        m = jnp.maximum(jnp.maximum(t[0], s62), s63)
        es = [jnp.exp(x - m) for x in vals]
        total = ((es[0] + es[1]) + (es[2] + es[3])) + (
            (es[4] + es[5]) + (es[6] + es[7]))
        r = 1.0 / total

        for k in range(K):
            ck = jnp.full((L,), k, dtype=jnp.int32)
            plsc.store_scatter(oidx, [row, ck], idxs[k])
            plsc.store_scatter(ogate, [row, ck], es[k] * r)
        return carry

    def chunk_body(ch, carry):
        crow = row_base + ch * CHUNK_ROWS
        pltpu.sync_copy(gates_hbm.at[pl.ds(crow, CHUNK_ROWS), :], buf)
        lax.fori_loop(0, GROUPS_PER_CHUNK, group_body, 0)
        pltpu.sync_copy(oidx, idx_hbm.at[pl.ds(crow, CHUNK_ROWS), :])
        pltpu.sync_copy(ogate, gate_hbm.at[pl.ds(crow, CHUNK_ROWS), :])
        return carry

    lax.fori_loop(0, CHUNKS, chunk_body, 0)


@jax.jit
def kernel(gates):
    out_idx, out_gate = _sc_topk(gates)
    return out_idx, out_gate
